# Initial kernel scaffold; baseline (speedup 1.0000x reference)
import jax
import jax.numpy as jnp
from jax.experimental import pallas as pl


def kernel(preds, targets):
    raise NotImplementedError("write your pallas kernel here")


# speedup vs baseline: 2.1248x; 2.1248x over previous
import functools

import jax
import jax.numpy as jnp
from jax import lax
from jax.experimental import pallas as pl
from jax.experimental.pallas import tpu as pltpu

ALPHA = 0.5
EVENT_W = 1.0
CENS_W = 1.0
EPS = 1e-8


def _partials_kernel(preds_ref, dur_row_ref, dur_col_ref, ev_row_ref,
                     ev_col_ref, out_ref, acc_ref, *, bi, ni2, num_t):
    c = pl.program_id(0)
    j = pl.program_id(1)
    i0 = (c * ni2 + j) * bi

    dur_row = dur_row_ref[...]
    maxd = jnp.max(dur_row)

    dur_i_row = dur_row_ref[:, pl.ds(i0, bi)]
    ev_i_row = ev_row_ref[:, pl.ds(i0, bi)]
    dur_i_col = dur_col_ref[pl.ds(i0, bi), :]
    ev_i_col = ev_col_ref[pl.ds(i0, bi), :]

    didx_row = (dur_i_row / maxd * (num_t - 1)).astype(jnp.int32)
    didx_col = (dur_i_col / maxd * (num_t - 1)).astype(jnp.int32)

    preds_i = preds_ref[pl.ds(i0, bi), :]
    onehot_col = (didx_col == lax.broadcasted_iota(
        jnp.int32, (bi, num_t), 1)).astype(jnp.float32)
    g_col = jnp.sum(preds_i * onehot_col, axis=1, keepdims=True)
    w_col = jnp.where(ev_i_col == 1.0, EVENT_W, CENS_W)
    lik_p = jnp.sum(-jnp.log(g_col + EPS) * ev_i_col * w_col)

    onehot_t = (lax.broadcasted_iota(jnp.int32, (num_t, bi), 0)
                == didx_row).astype(jnp.float32)
    m_cols = jnp.dot(preds_ref[...], onehot_t,
                     preferred_element_type=jnp.float32)

    d_blk = jnp.dot(preds_i, onehot_t,
                    preferred_element_type=jnp.float32)
    eye = (lax.broadcasted_iota(jnp.int32, (bi, bi), 0)
           == lax.broadcasted_iota(jnp.int32, (bi, bi), 1))
    g_row = jnp.sum(jnp.where(eye, d_blk, 0.0), axis=0,
                    keepdims=True)

    dur_col = dur_col_ref[...]
    land = (dur_col > dur_i_row) & (ev_i_row == 1.0)
    contrib = jnp.where(land, jnp.maximum(m_cols - g_row, 0.0), 0.0)
    rank_p = jnp.sum(contrib) * EVENT_W
    cnt_p = jnp.sum(land.astype(jnp.float32))

    @pl.when(j == 0)
    def _():
        acc_ref[0] = 0.0
        acc_ref[1] = 0.0
        acc_ref[2] = 0.0

    acc_ref[0] += lik_p
    acc_ref[1] += rank_p
    acc_ref[2] += cnt_p

    @pl.when(j == ni2 - 1)
    def _():
        lanes = lax.broadcasted_iota(jnp.int32, (1, 1, 128), 2)
        vec = jnp.where(lanes == 0, acc_ref[0],
                        jnp.where(lanes == 1, acc_ref[1],
                                  jnp.where(lanes == 2, acc_ref[2], 0.0)))
        out_ref[...] = vec


def _finish_kernel(p_ref, out_ref, *, n):
    lane = lax.broadcasted_iota(jnp.int32, (1, 128), 1)
    s = p_ref[0] + p_ref[1]
    lik_sum = jnp.sum(jnp.where(lane == 0, s, 0.0))
    rank_sum = jnp.sum(jnp.where(lane == 1, s, 0.0))
    cnt = jnp.sum(jnp.where(lane == 2, s, 0.0))
    rank = jnp.where(cnt > 0.0, rank_sum / jnp.maximum(cnt, 1.0), 0.0)
    res = ALPHA * (lik_sum / n) + (1.0 - ALPHA) * rank
    out_ref[...] = jnp.full((1, 128), res, dtype=jnp.float32)


def _deep_hit_loss(preds, targets, *, interpret=False):
    b, num_t = preds.shape
    bi = min(256, b // 2)
    ni2 = b // (2 * bi)

    dur = targets[:, 0]
    ev = targets[:, 1]
    dur_row = dur.reshape(1, b)
    dur_col = dur.reshape(b, 1)
    ev_row = ev.reshape(1, b)
    ev_col = ev.reshape(b, 1)

    partials = pl.pallas_call(
        functools.partial(_partials_kernel, bi=bi, ni2=ni2, num_t=num_t),
        grid=(2, ni2),
        in_specs=[
            pl.BlockSpec((b, num_t), lambda c, j: (0, 0)),
            pl.BlockSpec((1, b), lambda c, j: (0, 0)),
            pl.BlockSpec((b, 1), lambda c, j: (0, 0)),
            pl.BlockSpec((1, b), lambda c, j: (0, 0)),
            pl.BlockSpec((b, 1), lambda c, j: (0, 0)),
        ],
        out_specs=pl.BlockSpec((1, 1, 128), lambda c, j: (c, 0, 0)),
        out_shape=jax.ShapeDtypeStruct((2, 1, 128), jnp.float32),
        scratch_shapes=[pltpu.SMEM((4,), jnp.float32)],
        compiler_params=pltpu.CompilerParams(
            dimension_semantics=("parallel", "arbitrary"),
            vmem_limit_bytes=44 * 1024 * 1024,
        ),
        name="deep_hit_partials",
        interpret=interpret,
    )(preds, dur_row, dur_col, ev_row, ev_col)

    out = pl.pallas_call(
        functools.partial(_finish_kernel, n=float(b)),
        out_shape=jax.ShapeDtypeStruct((1, 128), jnp.float32),
        name="deep_hit_finish",
        interpret=interpret,
    )(partials)
    return out[0, 0]


def kernel(preds, targets):
    return _deep_hit_loss(preds, targets)


# speedup vs baseline: 2.6499x; 1.2471x over previous
import functools

import jax
import jax.numpy as jnp
from jax import lax
from jax.experimental import pallas as pl
from jax.experimental.pallas import tpu as pltpu

ALPHA = 0.5
EVENT_W = 1.0
CENS_W = 1.0
EPS = 1e-8


def _partials_kernel(preds_ref, dur_row_ref, dur_col_ref, ev_row_ref,
                     out_ref, acc_ref, *, bi, ni2, num_t):
    c = pl.program_id(0)
    j = pl.program_id(1)
    i0 = (c * ni2 + j) * bi

    dur_row = dur_row_ref[...]
    maxd = jnp.max(dur_row)

    dur_i_row = dur_row_ref[:, pl.ds(i0, bi)]
    ev_i_row = ev_row_ref[:, pl.ds(i0, bi)]

    didx_row = (dur_i_row / maxd * (num_t - 1)).astype(jnp.int32)

    onehot_t = (lax.broadcasted_iota(jnp.int32, (num_t, bi), 0)
                == didx_row).astype(jnp.float32)
    m_cols = jnp.dot(preds_ref[...], onehot_t,
                     preferred_element_type=jnp.float32)

    preds_i = preds_ref[pl.ds(i0, bi), :]
    d_blk = jnp.dot(preds_i, onehot_t,
                    preferred_element_type=jnp.float32)
    eye = (lax.broadcasted_iota(jnp.int32, (bi, bi), 0)
           == lax.broadcasted_iota(jnp.int32, (bi, bi), 1))
    g_row = jnp.sum(jnp.where(eye, d_blk, 0.0), axis=0,
                    keepdims=True)

    evf_row = jnp.where(ev_i_row == 1.0, 1.0, 0.0)
    w_row = jnp.where(ev_i_row == 1.0, EVENT_W, CENS_W)
    lik_p = jnp.sum(-jnp.log(g_row + EPS) * evf_row * w_row)

    g_eff = jnp.where(ev_i_row == 1.0, g_row, jnp.inf)
    dur_col = dur_col_ref[...]
    dgtf = jnp.where(dur_col > dur_i_row, 1.0, 0.0)
    contrib = jnp.maximum(m_cols - g_eff, 0.0) * dgtf
    rank_p = jnp.sum(contrib) * EVENT_W
    cnt_cols = jnp.sum(dgtf, axis=0, keepdims=True)
    cnt_p = jnp.sum(cnt_cols * evf_row)

    @pl.when(j == 0)
    def _():
        acc_ref[0] = 0.0
        acc_ref[1] = 0.0
        acc_ref[2] = 0.0

    acc_ref[0] += lik_p
    acc_ref[1] += rank_p
    acc_ref[2] += cnt_p

    @pl.when(j == ni2 - 1)
    def _():
        lanes = lax.broadcasted_iota(jnp.int32, (1, 1, 128), 2)
        vec = jnp.where(lanes == 0, acc_ref[0],
                        jnp.where(lanes == 1, acc_ref[1],
                                  jnp.where(lanes == 2, acc_ref[2], 0.0)))
        out_ref[...] = vec


def _finish_kernel(p_ref, out_ref, *, n):
    lane = lax.broadcasted_iota(jnp.int32, (1, 128), 1)
    s = p_ref[0] + p_ref[1]
    lik_sum = jnp.sum(jnp.where(lane == 0, s, 0.0))
    rank_sum = jnp.sum(jnp.where(lane == 1, s, 0.0))
    cnt = jnp.sum(jnp.where(lane == 2, s, 0.0))
    rank = jnp.where(cnt > 0.0, rank_sum / jnp.maximum(cnt, 1.0), 0.0)
    res = ALPHA * (lik_sum / n) + (1.0 - ALPHA) * rank
    out_ref[...] = jnp.full((1, 128), res, dtype=jnp.float32)


def _deep_hit_loss(preds, targets, *, interpret=False):
    b, num_t = preds.shape
    bi = min(256, b // 2)
    ni2 = b // (2 * bi)

    dur = targets[:, 0]
    ev = targets[:, 1]
    dur_row = dur.reshape(1, b)
    dur_col = dur.reshape(b, 1)
    ev_row = ev.reshape(1, b)

    partials = pl.pallas_call(
        functools.partial(_partials_kernel, bi=bi, ni2=ni2, num_t=num_t),
        grid=(2, ni2),
        in_specs=[
            pl.BlockSpec((b, num_t), lambda c, j: (0, 0)),
            pl.BlockSpec((1, b), lambda c, j: (0, 0)),
            pl.BlockSpec((b, 1), lambda c, j: (0, 0)),
            pl.BlockSpec((1, b), lambda c, j: (0, 0)),
        ],
        out_specs=pl.BlockSpec((1, 1, 128), lambda c, j: (c, 0, 0)),
        out_shape=jax.ShapeDtypeStruct((2, 1, 128), jnp.float32),
        scratch_shapes=[pltpu.SMEM((4,), jnp.float32)],
        compiler_params=pltpu.CompilerParams(
            dimension_semantics=("parallel", "arbitrary"),
            vmem_limit_bytes=44 * 1024 * 1024,
        ),
        name="deep_hit_partials",
        interpret=interpret,
    )(preds, dur_row, dur_col, ev_row)

    out = pl.pallas_call(
        functools.partial(_finish_kernel, n=float(b)),
        out_shape=jax.ShapeDtypeStruct((1, 128), jnp.float32),
        name="deep_hit_finish",
        interpret=interpret,
    )(partials)
    return out[0, 0]


def kernel(preds, targets):
    return _deep_hit_loss(preds, targets)
